# Initial kernel scaffold; baseline (speedup 1.0000x reference)
import jax
import jax.numpy as jnp
from jax.experimental import pallas as pl


def kernel(x, pos, batch, W1, b1, g1, be1, W2, b2):
    raise NotImplementedError("write your pallas kernel here")


# speedup vs baseline: 6.2058x; 6.2058x over previous
import functools

import jax
import jax.numpy as jnp
from jax.experimental import pallas as pl
from jax.experimental.pallas import tpu as pltpu

B_CLOUDS = 16
P = 2048
N = B_CLOUDS * P
D = 128
DIM = 64
K = 16
RR = 0.2 * 0.2
NEG_INF = float("-inf")
POS_INF = float("inf")

_RB = 256
_CB = 128


def _stage_a_body(x_ref, p0_ref, p1_ref, p2_ref, b_ref, w1a_ref, w1p_ref,
                  b1_ref, v_ref, q_ref, pix_ref):
    p0 = p0_ref[...]
    p1 = p1_ref[...]
    p2 = p2_ref[...]
    q = (p0 * w1p_ref[0:1, :] + p1 * w1p_ref[1:2, :] + p2 * w1p_ref[2:3, :])
    v = jnp.dot(x_ref[...], w1a_ref[...], preferred_element_type=jnp.float32)
    v_ref[...] = v + q + b1_ref[0:1, :]
    q_ref[...] = q
    rows = jnp.clip((p1 * DIM).astype(jnp.int32), 0, DIM - 1)
    cols = jnp.clip((p0 * DIM).astype(jnp.int32), 0, DIM - 1)
    pix_ref[...] = b_ref[...] * (DIM * DIM) + rows * DIM + cols


def _stage_a(x, p0, p1, p2, batch_col, w1a, w1p_pad, b1_row):
    nblk = N // _RB
    return pl.pallas_call(
        _stage_a_body,
        grid=(nblk,),
        in_specs=[
            pl.BlockSpec((_RB, D), lambda i: (i, 0)),
            pl.BlockSpec((_RB, 1), lambda i: (i, 0)),
            pl.BlockSpec((_RB, 1), lambda i: (i, 0)),
            pl.BlockSpec((_RB, 1), lambda i: (i, 0)),
            pl.BlockSpec((_RB, 1), lambda i: (i, 0)),
            pl.BlockSpec((D, D), lambda i: (0, 0)),
            pl.BlockSpec((8, D), lambda i: (0, 0)),
            pl.BlockSpec((1, D), lambda i: (0, 0)),
        ],
        out_specs=[
            pl.BlockSpec((_RB, D), lambda i: (i, 0)),
            pl.BlockSpec((_RB, D), lambda i: (i, 0)),
            pl.BlockSpec((_RB, 1), lambda i: (i, 0)),
        ],
        out_shape=[
            jax.ShapeDtypeStruct((N, D), jnp.float32),
            jax.ShapeDtypeStruct((N, D), jnp.float32),
            jax.ShapeDtypeStruct((N, 1), jnp.int32),
        ],
    )(x, p0, p1, p2, batch_col, w1a, w1p_pad, b1_row)


def _stage_b_body(pxt_ref, pyt_ref, pzt_ref, px_ref, py_ref, pz_ref,
                  nbr_ref, val_ref):
    b = pl.program_id(0)
    dx = pxt_ref[0] - px_ref[0]
    dy = pyt_ref[0] - py_ref[0]
    dz = pzt_ref[0] - pz_ref[0]
    d2 = (dx * dx + dy * dy) + dz * dz
    dm = jnp.where(d2 <= RR, d2, POS_INF)
    lanes = jax.lax.broadcasted_iota(jnp.int32, (_RB, P), 1)
    nbr_cols = []
    val_cols = []
    for _ in range(K):
        mn = jnp.min(dm, axis=1, keepdims=True)
        idx = jnp.min(jnp.where(dm == mn, lanes, P), axis=1, keepdims=True)
        ok = mn < POS_INF
        nbr_cols.append(jnp.minimum(idx, P - 1) + b * P)
        val_cols.append(jnp.where(ok, jnp.float32(0.0), NEG_INF))
        dm = jnp.where(lanes == idx, POS_INF, dm)
    nbr_ref[...] = jnp.concatenate(nbr_cols, axis=1)
    val_ref[...] = jnp.concatenate(val_cols, axis=1)


def _stage_b(pxt, pyt, pzt, px, py, pz):
    nrb = P // _RB
    return pl.pallas_call(
        _stage_b_body,
        grid=(B_CLOUDS, nrb),
        in_specs=[
            pl.BlockSpec((1, _RB, 1), lambda b, r: (b * (P // _RB) + r, 0, 0)),
            pl.BlockSpec((1, _RB, 1), lambda b, r: (b * (P // _RB) + r, 0, 0)),
            pl.BlockSpec((1, _RB, 1), lambda b, r: (b * (P // _RB) + r, 0, 0)),
            pl.BlockSpec((1, 1, P), lambda b, r: (b, 0, 0)),
            pl.BlockSpec((1, 1, P), lambda b, r: (b, 0, 0)),
            pl.BlockSpec((1, 1, P), lambda b, r: (b, 0, 0)),
        ],
        out_specs=[
            pl.BlockSpec((_RB, K), lambda b, r: (b * (P // _RB) + r, 0)),
            pl.BlockSpec((_RB, K), lambda b, r: (b * (P // _RB) + r, 0)),
        ],
        out_shape=[
            jax.ShapeDtypeStruct((N, K), jnp.int32),
            jax.ShapeDtypeStruct((N, K), jnp.float32),
        ],
    )(pxt, pyt, pzt, px, py, pz)


def _stage_c_body(e_ref, q_ref, val_ref, w2_ref, g1_ref, be1_ref, b2_ref,
                  m_ref):
    qb = q_ref[...]
    g1 = g1_ref[0:1, :]
    be1 = be1_ref[0:1, :]
    w2 = w2_ref[...]
    m = jnp.full((_CB, D), NEG_INF, jnp.float32)
    for k in range(K):
        gk = e_ref[k]
        a = jax.nn.relu((gk - qb) * g1 + be1)
        hk = jnp.dot(a, w2, preferred_element_type=jnp.float32)
        m = jnp.maximum(m, hk + val_ref[:, k:k + 1])
    m_ref[...] = m + b2_ref[0:1, :]


def _stage_c(e_kmaj, q, validf, w2, g1_row, be1_row, b2_row):
    nblk = N // _CB
    return pl.pallas_call(
        _stage_c_body,
        grid=(nblk,),
        in_specs=[
            pl.BlockSpec((K, _CB, D), lambda i: (0, i, 0)),
            pl.BlockSpec((_CB, D), lambda i: (i, 0)),
            pl.BlockSpec((_CB, K), lambda i: (i, 0)),
            pl.BlockSpec((D, D), lambda i: (0, 0)),
            pl.BlockSpec((1, D), lambda i: (0, 0)),
            pl.BlockSpec((1, D), lambda i: (0, 0)),
            pl.BlockSpec((1, D), lambda i: (0, 0)),
        ],
        out_specs=pl.BlockSpec((_CB, D), lambda i: (i, 0)),
        out_shape=jax.ShapeDtypeStruct((N, D), jnp.float32),
    )(e_kmaj, q, validf, w2, g1_row, be1_row, b2_row)


def kernel(x, pos, batch, W1, b1, g1, be1, W2, b2):
    p0 = pos[:, 0:1]
    p1 = pos[:, 1:2]
    p2 = pos[:, 2:3]
    w1a = W1[:D]
    w1p_pad = jnp.pad(W1[D:], ((0, 5), (0, 0)))
    b1_row = b1.reshape(1, D)

    v, q, pix = _stage_a(x, p0, p1, p2, batch.reshape(N, 1), w1a, w1p_pad,
                         b1_row)

    qshape = (N // _RB, _RB, 1)
    cshape = (B_CLOUDS, 1, P)
    nbr, validf = _stage_b(p0.reshape(qshape), p1.reshape(qshape),
                           p2.reshape(qshape), pos[:, 0].reshape(cshape),
                           pos[:, 1].reshape(cshape), pos[:, 2].reshape(cshape))

    e_kmaj = jnp.take(v, nbr.T, axis=0)

    m = _stage_c(e_kmaj, q, validf, W2, g1.reshape(1, D), be1.reshape(1, D),
                 b2.reshape(1, D))

    out = jnp.full((B_CLOUDS * DIM * DIM, D), NEG_INF, jnp.float32)
    out = out.at[pix[:, 0]].max(m)
    out = jnp.where(jnp.isneginf(out), jnp.float32(0.0), out)
    return out.reshape(B_CLOUDS, DIM, DIM, D).transpose(0, 3, 1, 2)


# speedup vs baseline: 11.3731x; 1.8327x over previous
import functools

import jax
import jax.numpy as jnp
from jax import lax
from jax.experimental import pallas as pl
from jax.experimental.pallas import tpu as pltpu
from jax.experimental.pallas import tpu_sc as plsc

B_CLOUDS = 16
P = 2048
N = B_CLOUDS * P
D = 128
DIM = 64
K = 16
RR = 0.2 * 0.2
NEG_INF = float("-inf")
POS_INF = float("inf")

_RB = 256
_CB = 128


def _stage_a_body(x_ref, p0_ref, p1_ref, p2_ref, b_ref, w1a_ref, w1p_ref,
                  b1_ref, v_ref, q_ref, pix_ref):
    p0 = p0_ref[...]
    p1 = p1_ref[...]
    p2 = p2_ref[...]
    q = (p0 * w1p_ref[0:1, :] + p1 * w1p_ref[1:2, :] + p2 * w1p_ref[2:3, :])
    v = jnp.dot(x_ref[...], w1a_ref[...], preferred_element_type=jnp.float32)
    v_ref[...] = v + q + b1_ref[0:1, :]
    q_ref[...] = q
    rows = jnp.clip((p1 * DIM).astype(jnp.int32), 0, DIM - 1)
    cols = jnp.clip((p0 * DIM).astype(jnp.int32), 0, DIM - 1)
    pix_ref[...] = b_ref[...] * (DIM * DIM) + rows * DIM + cols


def _stage_a(x, p0, p1, p2, batch_col, w1a, w1p_pad, b1_row):
    nblk = N // _RB
    return pl.pallas_call(
        _stage_a_body,
        grid=(nblk,),
        in_specs=[
            pl.BlockSpec((_RB, D), lambda i: (i, 0)),
            pl.BlockSpec((_RB, 1), lambda i: (i, 0)),
            pl.BlockSpec((_RB, 1), lambda i: (i, 0)),
            pl.BlockSpec((_RB, 1), lambda i: (i, 0)),
            pl.BlockSpec((_RB, 1), lambda i: (i, 0)),
            pl.BlockSpec((D, D), lambda i: (0, 0)),
            pl.BlockSpec((8, D), lambda i: (0, 0)),
            pl.BlockSpec((1, D), lambda i: (0, 0)),
        ],
        out_specs=[
            pl.BlockSpec((_RB, D), lambda i: (i, 0)),
            pl.BlockSpec((_RB, D), lambda i: (i, 0)),
            pl.BlockSpec((_RB, 1), lambda i: (i, 0)),
        ],
        out_shape=[
            jax.ShapeDtypeStruct((N, D), jnp.float32),
            jax.ShapeDtypeStruct((N, D), jnp.float32),
            jax.ShapeDtypeStruct((N, 1), jnp.int32),
        ],
    )(x, p0, p1, p2, batch_col, w1a, w1p_pad, b1_row)


def _stage_b_body(pxt_ref, pyt_ref, pzt_ref, px_ref, py_ref, pz_ref,
                  nbr_ref, val_ref):
    b = pl.program_id(0)
    dx = pxt_ref[0] - px_ref[0]
    dy = pyt_ref[0] - py_ref[0]
    dz = pzt_ref[0] - pz_ref[0]
    d2 = (dx * dx + dy * dy) + dz * dz
    dm = jnp.where(d2 <= RR, d2, POS_INF)
    lanes = jax.lax.broadcasted_iota(jnp.int32, (_RB, P), 1)
    nbr_cols = []
    val_cols = []
    for _ in range(K):
        mn = jnp.min(dm, axis=1, keepdims=True)
        idx = jnp.min(jnp.where(dm == mn, lanes, P), axis=1, keepdims=True)
        ok = mn < POS_INF
        nbr_cols.append(jnp.minimum(idx, P - 1) + b * P)
        val_cols.append(jnp.where(ok, jnp.float32(0.0), NEG_INF))
        dm = jnp.where(lanes == idx, POS_INF, dm)
    nbr_ref[...] = jnp.concatenate(nbr_cols, axis=1)
    val_ref[...] = jnp.concatenate(val_cols, axis=1)


def _stage_b(pxt, pyt, pzt, px, py, pz):
    nrb = P // _RB
    return pl.pallas_call(
        _stage_b_body,
        grid=(B_CLOUDS, nrb),
        in_specs=[
            pl.BlockSpec((1, _RB, 1), lambda b, r: (b * (P // _RB) + r, 0, 0)),
            pl.BlockSpec((1, _RB, 1), lambda b, r: (b * (P // _RB) + r, 0, 0)),
            pl.BlockSpec((1, _RB, 1), lambda b, r: (b * (P // _RB) + r, 0, 0)),
            pl.BlockSpec((1, 1, P), lambda b, r: (b, 0, 0)),
            pl.BlockSpec((1, 1, P), lambda b, r: (b, 0, 0)),
            pl.BlockSpec((1, 1, P), lambda b, r: (b, 0, 0)),
        ],
        out_specs=[
            pl.BlockSpec((_RB, K), lambda b, r: (b * (P // _RB) + r, 0)),
            pl.BlockSpec((_RB, K), lambda b, r: (b * (P // _RB) + r, 0)),
        ],
        out_shape=[
            jax.ShapeDtypeStruct((N, K), jnp.int32),
            jax.ShapeDtypeStruct((N, K), jnp.float32),
        ],
    )(pxt, pyt, pzt, px, py, pz)


_EDGES = K * N


def _make_sc_gather():
    info = plsc.get_sparse_core_info()
    nw = info.num_cores * info.num_subcores
    epw = _EDGES // nw
    gch = 128
    nch = epw // gch
    mesh = plsc.VectorSubcoreMesh(core_axis_name="c", subcore_axis_name="s")

    @functools.partial(
        pl.kernel, mesh=mesh,
        out_type=jax.ShapeDtypeStruct((_EDGES, D), jnp.float32),
        scratch_types=[
            pltpu.VMEM((epw,), jnp.int32),
            pltpu.VMEM((gch, D), jnp.float32),
            pltpu.VMEM((gch, D), jnp.float32),
            pltpu.SemaphoreType.DMA,
            pltpu.SemaphoreType.DMA,
        ],
    )
    def sc_gather(v_hbm, idx_hbm, out_hbm, idx_v, buf0, buf1, sem0, sem1):
        wid = lax.axis_index("s") * info.num_cores + lax.axis_index("c")
        base = wid * epw
        pltpu.sync_copy(idx_hbm.at[pl.ds(base, epw)], idx_v)
        bufs = (buf0, buf1)
        sems = (sem0, sem1)

        def _start(i, slot):
            pltpu.async_copy(v_hbm.at[idx_v.at[pl.ds(i * gch, gch)]],
                             bufs[slot], sems[slot])

        _start(0, 0)
        def step(i, carry):
            for slot in range(2):
                @pl.when(i % 2 == slot)
                def _():
                    @pl.when(i + 1 < nch)
                    def _():
                        _start(i + 1, 1 - slot)
                    pltpu.make_async_copy(
                        v_hbm.at[idx_v.at[pl.ds(i * gch, gch)]],
                        bufs[slot], sems[slot]).wait()
                    pltpu.sync_copy(bufs[slot],
                                    out_hbm.at[pl.ds(base + i * gch, gch)])
            return carry

        lax.fori_loop(0, nch, step, 0)

    return sc_gather


_sc_gather = _make_sc_gather()


def _stage_c_body(e_ref, q_ref, val_ref, w2_ref, g1_ref, be1_ref, b2_ref,
                  m_ref):
    qb = q_ref[...]
    g1 = g1_ref[0:1, :]
    be1 = be1_ref[0:1, :]
    w2 = w2_ref[...]
    m = jnp.full((_CB, D), NEG_INF, jnp.float32)
    for k in range(K):
        gk = e_ref[k]
        a = jax.nn.relu((gk - qb) * g1 + be1)
        hk = jnp.dot(a, w2, preferred_element_type=jnp.float32)
        m = jnp.maximum(m, hk + val_ref[:, k:k + 1])
    m_ref[...] = m + b2_ref[0:1, :]


def _stage_c(e_kmaj, q, validf, w2, g1_row, be1_row, b2_row):
    nblk = N // _CB
    return pl.pallas_call(
        _stage_c_body,
        grid=(nblk,),
        in_specs=[
            pl.BlockSpec((K, _CB, D), lambda i: (0, i, 0)),
            pl.BlockSpec((_CB, D), lambda i: (i, 0)),
            pl.BlockSpec((_CB, K), lambda i: (i, 0)),
            pl.BlockSpec((D, D), lambda i: (0, 0)),
            pl.BlockSpec((1, D), lambda i: (0, 0)),
            pl.BlockSpec((1, D), lambda i: (0, 0)),
            pl.BlockSpec((1, D), lambda i: (0, 0)),
        ],
        out_specs=pl.BlockSpec((_CB, D), lambda i: (i, 0)),
        out_shape=jax.ShapeDtypeStruct((N, D), jnp.float32),
    )(e_kmaj, q, validf, w2, g1_row, be1_row, b2_row)


def kernel(x, pos, batch, W1, b1, g1, be1, W2, b2):
    p0 = pos[:, 0:1]
    p1 = pos[:, 1:2]
    p2 = pos[:, 2:3]
    w1a = W1[:D]
    w1p_pad = jnp.pad(W1[D:], ((0, 5), (0, 0)))
    b1_row = b1.reshape(1, D)

    v, q, pix = _stage_a(x, p0, p1, p2, batch.reshape(N, 1), w1a, w1p_pad,
                         b1_row)

    qshape = (N // _RB, _RB, 1)
    cshape = (B_CLOUDS, 1, P)
    nbr, validf = _stage_b(p0.reshape(qshape), p1.reshape(qshape),
                           p2.reshape(qshape), pos[:, 0].reshape(cshape),
                           pos[:, 1].reshape(cshape), pos[:, 2].reshape(cshape))

    e_kmaj = _sc_gather(v, nbr.T.reshape(_EDGES)).reshape(K, N, D)

    m = _stage_c(e_kmaj, q, validf, W2, g1.reshape(1, D), be1.reshape(1, D),
                 b2.reshape(1, D))

    out = jnp.full((B_CLOUDS * DIM * DIM, D), NEG_INF, jnp.float32)
    out = out.at[pix[:, 0]].max(m)
    out = jnp.where(jnp.isneginf(out), jnp.float32(0.0), out)
    return out.reshape(B_CLOUDS, DIM, DIM, D).transpose(0, 3, 1, 2)
